# Initial kernel scaffold; baseline (speedup 1.0000x reference)
import jax
import jax.numpy as jnp
from jax.experimental import pallas as pl


def kernel(x, edge_index, edge_attr, w1, root1, b1, w2, root2, b2, Wf, bf):
    raise NotImplementedError("write your pallas kernel here")


# speedup vs baseline: 3.3350x; 3.3350x over previous
import functools

import jax
import jax.numpy as jnp
from jax import lax
from jax.experimental import pallas as pl
from jax.experimental.pallas import tpu as pltpu
from jax.experimental.pallas import tpu_sc as plsc

N_NODES = 10000
N_EDGES = 160000
KS = 5
K = KS * KS
IN_CH = 128
OUT_CH = 64
PW = 2 * OUT_CH

NC = 2
NS = 16
L = 16
NW = NC * NS

CHUNK = 64
CHUNKS = 80
E_W = CHUNK * CHUNKS
E_PAD = NW * E_W
WMSG = 80
NROWS = 10240
ROWS_T = NROWS // NS

MBLK = 400


def _pair_mm_body(x_ref, w_ref, o_ref):
    o_ref[0] = jnp.dot(x_ref[...], w_ref[0],
                       preferred_element_type=jnp.float32)


def _pair_table(x, wpair):
    m, kd = x.shape
    return pl.pallas_call(
        _pair_mm_body,
        grid=(K, m // MBLK),
        in_specs=[
            pl.BlockSpec((MBLK, kd), lambda p, i: (i, 0)),
            pl.BlockSpec((1, kd, PW), lambda p, i: (p, 0, 0)),
        ],
        out_specs=pl.BlockSpec((1, MBLK, PW), lambda p, i: (p, i, 0)),
        out_shape=jax.ShapeDtypeStruct((K, m, PW), jnp.float32),
    )(x, wpair).reshape(K * m, PW)


def _basis_body(src_ref, ea0_ref, ea1_ref, ga_ref, gb_ref,
                b0_ref, b1_ref, b2_ref, b3_ref):
    v0 = ea0_ref[...] * (KS - 1.0)
    v1 = ea1_ref[...] * (KS - 1.0)
    f0 = jnp.minimum(v0.astype(jnp.int32), KS - 2)
    f1 = jnp.minimum(v1.astype(jnp.int32), KS - 2)
    r0 = v0 - f0.astype(jnp.float32)
    r1 = v1 - f1.astype(jnp.float32)
    ra = (f0 + KS * f1) * N_NODES + src_ref[...]
    ga_ref[...] = ra
    gb_ref[...] = ra + KS * N_NODES
    b0_ref[...] = (1.0 - r0) * (1.0 - r1)
    b1_ref[...] = r0 * (1.0 - r1)
    b2_ref[...] = (1.0 - r0) * r1
    b3_ref[...] = r0 * r1


def _basis(src, ea0, ea1):
    rows = E_PAD // 128
    blk = 16
    grid = rows // blk
    s2 = src.reshape(rows, 128)
    e0 = ea0.reshape(rows, 128)
    e1 = ea1.reshape(rows, 128)
    espec = pl.BlockSpec((blk, 128), lambda i: (i, 0))
    out = pl.pallas_call(
        _basis_body,
        grid=(grid,),
        in_specs=[espec, espec, espec],
        out_specs=[espec] * 6,
        out_shape=[jax.ShapeDtypeStruct((rows, 128), jnp.int32)] * 2
        + [jax.ShapeDtypeStruct((rows, 128), jnp.float32)] * 4,
    )(s2, e0, e1)
    return [a.reshape(NW, E_W) for a in out]


def _post1_body(a0_ref, a1_ref, x_ref, r1_ref, b1_ref, h1_ref):
    s = a0_ref[:, :OUT_CH] + a1_ref[:, :OUT_CH]
    deg = a0_ref[:, OUT_CH:OUT_CH + 1] + a1_ref[:, OUT_CH:OUT_CH + 1]
    agg = s / jnp.maximum(deg, 1.0)
    h1_ref[...] = jnp.maximum(
        agg + jnp.dot(x_ref[...], r1_ref[...],
                      preferred_element_type=jnp.float32) + b1_ref[...], 0.0)


def _post1(a0, a1, x, root1, b1):
    grid = N_NODES // MBLK
    return pl.pallas_call(
        _post1_body,
        grid=(grid,),
        in_specs=[
            pl.BlockSpec((MBLK, WMSG), lambda i: (i, 0)),
            pl.BlockSpec((MBLK, WMSG), lambda i: (i, 0)),
            pl.BlockSpec((MBLK, IN_CH), lambda i: (i, 0)),
            pl.BlockSpec((IN_CH, OUT_CH), lambda i: (0, 0)),
            pl.BlockSpec((1, OUT_CH), lambda i: (0, 0)),
        ],
        out_specs=pl.BlockSpec((MBLK, OUT_CH), lambda i: (i, 0)),
        out_shape=jax.ShapeDtypeStruct((N_NODES, OUT_CH), jnp.float32),
    )(a0, a1, x, root1, b1)


def _post2_body(a0_ref, a1_ref, x_ref, h1_ref, r2_ref, b2_ref,
                wfx_ref, wf1_ref, wf2_ref, bf_ref, o_ref):
    s = a0_ref[:, :OUT_CH] + a1_ref[:, :OUT_CH]
    deg = a0_ref[:, OUT_CH:OUT_CH + 1] + a1_ref[:, OUT_CH:OUT_CH + 1]
    agg = s / jnp.maximum(deg, 1.0)
    h2 = jnp.maximum(
        agg + jnp.dot(h1_ref[...], r2_ref[...],
                      preferred_element_type=jnp.float32) + b2_ref[...], 0.0)
    o_ref[...] = (
        jnp.dot(x_ref[...], wfx_ref[...], preferred_element_type=jnp.float32)
        + jnp.dot(h1_ref[...], wf1_ref[...],
                  preferred_element_type=jnp.float32)
        + jnp.dot(h2, wf2_ref[...], preferred_element_type=jnp.float32)
        + bf_ref[...])


def _post2(a0, a1, x, h1, root2, b2, wfx, wf1, wf2, bf):
    grid = N_NODES // MBLK
    return pl.pallas_call(
        _post2_body,
        grid=(grid,),
        in_specs=[
            pl.BlockSpec((MBLK, WMSG), lambda i: (i, 0)),
            pl.BlockSpec((MBLK, WMSG), lambda i: (i, 0)),
            pl.BlockSpec((MBLK, IN_CH), lambda i: (i, 0)),
            pl.BlockSpec((MBLK, OUT_CH), lambda i: (i, 0)),
            pl.BlockSpec((OUT_CH, OUT_CH), lambda i: (0, 0)),
            pl.BlockSpec((1, OUT_CH), lambda i: (0, 0)),
            pl.BlockSpec((IN_CH, OUT_CH), lambda i: (0, 0)),
            pl.BlockSpec((OUT_CH, OUT_CH), lambda i: (0, 0)),
            pl.BlockSpec((OUT_CH, OUT_CH), lambda i: (0, 0)),
            pl.BlockSpec((1, OUT_CH), lambda i: (0, 0)),
        ],
        out_specs=pl.BlockSpec((MBLK, OUT_CH), lambda i: (i, 0)),
        out_shape=jax.ShapeDtypeStruct((N_NODES, OUT_CH), jnp.float32),
    )(a0, a1, x, h1, root2, b2, wfx, wf1, wf2, bf)


def _sc_body(table, ga_h, gb_h, b0_h, b1_h, b2_h, b3_h, dst_h, out_h,
             ga_v, gb_v, b0_v, b1_v, b2_v, b3_v, dst_v, rows_v, msg_v,
             cnt_s, acc_sh, sem):
    cid = lax.axis_index("c")
    sid = lax.axis_index("s")
    wid = cid * NS + sid

    @pl.when(sid == 0)
    def _():
        cnt_s[0] = 0

    pltpu.sync_copy(ga_h.at[wid], ga_v)
    pltpu.sync_copy(gb_h.at[wid], gb_v)
    pltpu.sync_copy(b0_h.at[wid], b0_v)
    pltpu.sync_copy(b1_h.at[wid], b1_v)
    pltpu.sync_copy(b2_h.at[wid], b2_v)
    pltpu.sync_copy(b3_h.at[wid], b3_v)
    pltpu.sync_copy(dst_h.at[wid], dst_v)

    zvec = jnp.zeros((L,), jnp.float32)
    cvec = jnp.where(lax.iota(jnp.int32, L) == 0, 1.0, 0.0).astype(jnp.float32)

    def zmsg(i, _):
        for v in range(WMSG // L):
            msg_v[i, pl.ds(v * L, L)] = zvec
        return 0

    lax.fori_loop(0, CHUNK, zmsg, 0)

    row0 = sid * ROWS_T
    for j in range(ROWS_T // CHUNK):
        pltpu.sync_copy(msg_v,
                        acc_sh.at[pl.ds(row0 + j * CHUNK, CHUNK)])
    plsc.subcore_barrier()

    def ctail(i, _):
        msg_v[i, pl.ds(OUT_CH, L)] = cvec
        return 0

    lax.fori_loop(0, CHUNK, ctail, 0)

    bvs = [b0_v, b1_v, b2_v, b3_v]

    def chunk_body(c, _):
        co = c * CHUNK

        cpa = pltpu.async_copy(table.at[ga_v.at[pl.ds(co, CHUNK)]],
                               rows_v.at[pl.ds(0, CHUNK)], sem)
        cpb = pltpu.async_copy(table.at[gb_v.at[pl.ds(co, CHUNK)]],
                               rows_v.at[pl.ds(CHUNK, CHUNK)], sem)
        cpa.wait()
        cpb.wait()

        for j in range(CHUNK // L):
            o = j * L
            bvecs = [bvs[s][pl.ds(co + o, L)] for s in range(4)]
            for e in range(L):
                i = o + e
                accs = [None] * (OUT_CH // L)
                for s in range(4):
                    bs = jnp.full((L,), bvecs[s][e])
                    half = (s & 1) * OUT_CH
                    for v in range(OUT_CH // L):
                        r = rows_v[(s >> 1) * CHUNK + i,
                                   pl.ds(half + v * L, L)]
                        if s == 0:
                            accs[v] = r * bs
                        else:
                            accs[v] = accs[v] + r * bs
                for v in range(OUT_CH // L):
                    msg_v[i, pl.ds(v * L, L)] = accs[v]

        pltpu.sync_copy(msg_v, acc_sh.at[dst_v.at[c]], add=True)
        return 0

    lax.fori_loop(0, CHUNKS, chunk_body, 0)

    old = plsc.fetch_and_add(cnt_s.at[0], 1, subcore_id=0)

    @pl.when(old == NS - 1)
    def _():
        pltpu.sync_copy(acc_sh, out_h.at[cid])


@functools.partial(
    pl.kernel,
    out_type=jax.ShapeDtypeStruct((NC, NROWS, WMSG), jnp.float32),
    mesh=plsc.VectorSubcoreMesh(core_axis_name="c", subcore_axis_name="s",
                                num_cores=NC, num_subcores=NS),
    compiler_params=pltpu.CompilerParams(use_tc_tiling_on_sc=False),
    scratch_types=[
        pltpu.VMEM((E_W,), jnp.int32),
        pltpu.VMEM((E_W,), jnp.int32),
        pltpu.VMEM((E_W,), jnp.float32),
        pltpu.VMEM((E_W,), jnp.float32),
        pltpu.VMEM((E_W,), jnp.float32),
        pltpu.VMEM((E_W,), jnp.float32),
        pltpu.VMEM((CHUNKS, CHUNK), jnp.int32),
        pltpu.VMEM((2 * CHUNK, PW), jnp.float32),
        pltpu.VMEM((CHUNK, WMSG), jnp.float32),
        pltpu.SMEM((8,), jnp.int32),
        pltpu.VMEM_SHARED((NROWS, WMSG), jnp.float32),
        pltpu.SemaphoreType.DMA,
    ],
)
def _sc_pass(table, ga_h, gb_h, b0_h, b1_h, b2_h, b3_h, dst_h, out_h,
             *scratch):
    _sc_body(table, ga_h, gb_h, b0_h, b1_h, b2_h, b3_h, dst_h, out_h,
             *scratch)


def _pair_weights(w):
    i0 = jnp.arange(K, dtype=jnp.int32) % KS
    pnext = (i0 + 1) % KS + (jnp.arange(K, dtype=jnp.int32) // KS) * KS
    return jnp.concatenate([w, w[pnext]], axis=-1)


def kernel(x, edge_index, edge_attr, w1, root1, b1, w2, root2, b2, Wf, bf):
    npad = E_PAD - N_EDGES
    src = jnp.concatenate(
        [edge_index[0], jnp.zeros((npad,), jnp.int32)])
    pad_dst = N_NODES + jnp.arange(npad, dtype=jnp.int32) % (NROWS - N_NODES)
    dst = jnp.concatenate([edge_index[1], pad_dst])
    dst3 = dst.reshape(NW, CHUNKS, CHUNK)
    ea_pad = jnp.concatenate(
        [edge_attr, jnp.zeros((npad, 2), jnp.float32)], axis=0)

    ga, gb, b0, b1x, b2x, b3 = _basis(src, ea_pad[:, 0], ea_pad[:, 1])

    table1 = _pair_table(x, _pair_weights(w1))
    parts1 = _sc_pass(table1, ga, gb, b0, b1x, b2x, b3, dst3)
    h1 = _post1(parts1[0, :N_NODES], parts1[1, :N_NODES],
                x, root1, b1.reshape(1, OUT_CH))
    table2 = _pair_table(h1, _pair_weights(w2))
    parts2 = _sc_pass(table2, ga, gb, b0, b1x, b2x, b3, dst3)
    out = _post2(parts2[0, :N_NODES], parts2[1, :N_NODES], x, h1,
                 root2, b2.reshape(1, OUT_CH),
                 Wf[:IN_CH], Wf[IN_CH:IN_CH + OUT_CH], Wf[IN_CH + OUT_CH:],
                 bf.reshape(1, OUT_CH))
    return out
